# Initial kernel scaffold; baseline (speedup 1.0000x reference)
import jax
import jax.numpy as jnp
from jax.experimental import pallas as pl


def kernel(xyz, point_features, W0, b0):
    raise NotImplementedError("write your pallas kernel here")


# speedup vs baseline: 1.9766x; 1.9766x over previous
import jax
import jax.numpy as jnp
from jax.experimental import pallas as pl
from jax.experimental.pallas import tpu as pltpu

_GRID = 32
_NUM_VOX = _GRID * _GRID * _GRID
_BN_EPS = 1e-5
_NEG_INF = float("-inf")


def _stats_body(xyz_ref, pf_ref, w_ref, b_ref, acc_ref):
    b = pl.program_id(0)
    i = pl.program_id(1)

    @pl.when(jnp.logical_and(b == 0, i == 0))
    def _init():
        acc_ref[...] = jnp.zeros_like(acc_ref)

    xyz = xyz_ref[0]
    pf = pf_ref[0]
    w3 = w_ref[0:3, :]
    wp = w_ref[3:131, :]
    h = jax.lax.dot_general(xyz, w3, (((0,), (0,)), ((), ())),
                            preferred_element_type=jnp.float32)
    h += jax.lax.dot_general(pf, wp, (((0,), (0,)), ((), ())),
                             preferred_element_type=jnp.float32)
    h += b_ref[...]
    h = jnp.maximum(h, 0.0)
    acc_ref[0:1, :] += jnp.sum(h, axis=0, keepdims=True)
    acc_ref[1:2, :] += jnp.sum(h * h, axis=0, keepdims=True)


def _scatter_body(idx_ref, xyz_ref, pf_ref, w_ref, b_ref, mean_ref, rstd_ref,
                  out_ref, h_scr, t0, t1, t2, sem):
    bb = pl.program_id(0)
    i = pl.program_id(1)
    n = xyz_ref.shape[2]
    nchunks = pl.num_programs(1)

    @pl.when(i == 0)
    def _init():
        t0[...] = jnp.full_like(t0, _NEG_INF)
        t1[...] = jnp.full_like(t1, _NEG_INF)
        t2[...] = jnp.full_like(t2, _NEG_INF)

    xyz = xyz_ref[0]
    pf = pf_ref[0]
    w3 = w_ref[0:3, :]
    wp = w_ref[3:131, :]
    h = jax.lax.dot_general(xyz, w3, (((0,), (0,)), ((), ())),
                            preferred_element_type=jnp.float32)
    h += jax.lax.dot_general(pf, wp, (((0,), (0,)), ((), ())),
                             preferred_element_type=jnp.float32)
    h += b_ref[...]
    h = jnp.maximum(h, 0.0)
    h = (h - mean_ref[...]) * rstd_ref[...]
    h_scr[...] = h

    def body(g, _):
        p = g * 8
        v8 = h_scr[pl.ds(p, 8), :]
        for j in range(8):
            v = idx_ref[0, 0, p + j]
            val = v8[j:j + 1, :]
            if j % 3 == 0:
                row = t0[pl.ds(v, 1), :]
                t0[pl.ds(v, 1), :] = jnp.maximum(row, val)
            elif j % 3 == 1:
                row = t1[pl.ds(v, 1), :]
                t1[pl.ds(v, 1), :] = jnp.maximum(row, val)
            else:
                row = t2[pl.ds(v, 1), :]
                t2[pl.ds(v, 1), :] = jnp.maximum(row, val)
        return _

    jax.lax.fori_loop(0, n // 8, body, 0, unroll=4)

    @pl.when(i == nchunks - 1)
    def _fin():
        mx = jnp.maximum(jnp.maximum(t0[...], t1[...]), t2[...])
        t0[...] = jnp.where(jnp.isneginf(mx), 0.0, mx)
        cp = pltpu.make_async_copy(t0, out_ref.at[bb], sem)
        cp.start()
        cp.wait()


def kernel(xyz, point_features, W0, b0):
    B, _, N = xyz.shape
    C = W0.shape[1]

    g = jnp.float32(_GRID - 1)
    ii = jnp.round(xyz * g).astype(jnp.int32)
    ii = jnp.clip(ii, 0, _GRID - 1)
    flat_idx = ii[:, 0, :] * (_GRID * _GRID) + ii[:, 1, :] * _GRID + ii[:, 2, :]

    b2 = b0.reshape(1, C)

    n1 = 8192
    nc1 = N // n1
    stats = pl.pallas_call(
        _stats_body,
        grid=(B, nc1),
        in_specs=[
            pl.BlockSpec((1, 3, n1), lambda b, i: (b, 0, i)),
            pl.BlockSpec((1, 128, n1), lambda b, i: (b, 0, i)),
            pl.BlockSpec((131, C), lambda b, i: (0, 0)),
            pl.BlockSpec((1, C), lambda b, i: (0, 0)),
        ],
        out_specs=pl.BlockSpec((2, C), lambda b, i: (0, 0)),
        out_shape=jax.ShapeDtypeStruct((2, C), jnp.float32),
    )(xyz, point_features, W0, b2)

    m = jnp.float32(B * N)
    mean = stats[0:1, :] / m
    var = stats[1:2, :] / m - mean * mean
    rstd = jax.lax.rsqrt(var + _BN_EPS)

    n2 = 4096
    nc2 = N // n2
    idx3 = flat_idx.reshape(B * nc2, 1, n2)
    seg = pl.pallas_call(
        _scatter_body,
        grid=(B, nc2),
        in_specs=[
            pl.BlockSpec((1, 1, n2), lambda b, i, _nc=nc2: (b * _nc + i, 0, 0),
                         memory_space=pltpu.SMEM),
            pl.BlockSpec((1, 3, n2), lambda b, i: (b, 0, i)),
            pl.BlockSpec((1, 128, n2), lambda b, i: (b, 0, i)),
            pl.BlockSpec((131, C), lambda b, i: (0, 0)),
            pl.BlockSpec((1, C), lambda b, i: (0, 0)),
            pl.BlockSpec((1, C), lambda b, i: (0, 0)),
            pl.BlockSpec((1, C), lambda b, i: (0, 0)),
        ],
        out_specs=pl.BlockSpec(memory_space=pl.ANY),
        out_shape=jax.ShapeDtypeStruct((B, _NUM_VOX, C), jnp.float32),
        scratch_shapes=[pltpu.VMEM((n2, C), jnp.float32),
                        pltpu.VMEM((_NUM_VOX, C), jnp.float32),
                        pltpu.VMEM((_NUM_VOX, C), jnp.float32),
                        pltpu.VMEM((_NUM_VOX, C), jnp.float32),
                        pltpu.SemaphoreType.DMA],
    )(idx3, xyz, point_features, W0, b2, mean, rstd)

    return jnp.transpose(seg, (0, 2, 1)).reshape(B, C, _GRID, _GRID, _GRID)
